# baseline (speedup 1.0000x reference)
import functools

import jax
import jax.numpy as jnp
from jax import lax
from jax.experimental import pallas as pl
from jax.experimental.pallas import tpu as pltpu
from jax.experimental.pallas import tpu_sc as plsc

N_ROWS = 1000000
PARAM_DIM = 64
BATCH = 16384

_NC = 2
_NS = 16
_NW = _NC * _NS
_B_PER_W = BATCH // _NW

_mesh = plsc.VectorSubcoreMesh(core_axis_name="c", subcore_axis_name="s")


@functools.partial(
    pl.kernel,
    mesh=_mesh,
    out_type=jax.ShapeDtypeStruct((BATCH, PARAM_DIM), jnp.float32),
    scratch_types=[
        pltpu.VMEM((_B_PER_W,), jnp.int32),
        pltpu.VMEM((_B_PER_W, PARAM_DIM), jnp.float32),
        pltpu.SemaphoreType.DMA,
    ],
    compiler_params=pltpu.CompilerParams(use_tc_tiling_on_sc=False),
)
def _gather_kernel(x_hbm, table_hbm, out_hbm, idx_v, rows_v, sem):
    wid = lax.axis_index("s") * _NC + lax.axis_index("c")
    base = wid * _B_PER_W
    pltpu.sync_copy(x_hbm.at[pl.ds(base, _B_PER_W)], idx_v)
    pltpu.async_copy(table_hbm.at[idx_v], rows_v, sem).wait()
    pltpu.sync_copy(rows_v, out_hbm.at[pl.ds(base, _B_PER_W)])


def kernel(x, table):
    return _gather_kernel(x, table)


# speedup vs baseline: 2.8489x; 2.8489x over previous
import functools

import jax
import jax.numpy as jnp
from jax import lax
from jax.experimental import pallas as pl
from jax.experimental.pallas import tpu as pltpu
from jax.experimental.pallas import tpu_sc as plsc

N_ROWS = 1000000
PARAM_DIM = 64
BATCH = 16384

_NC = 2
_NS = 16
_NW = _NC * _NS

_LANE = 16
_BLK = 128
_NBLOCKS_FULL = 7812
_N_MAIN = _NBLOCKS_FULL * _BLK
_BPW = 245
_NBUF = 4
_STEPS = (_BPW + _NBUF - 1) // _NBUF
_LISTCAP = 2048
_NDUMP = 512
_OUT_ROWS = BATCH + _NDUMP
_SENTINEL = 0x7FFFFF00

_mesh = plsc.VectorSubcoreMesh(core_axis_name="c", subcore_axis_name="s")


@functools.partial(
    pl.kernel,
    mesh=_mesh,
    out_type=jax.ShapeDtypeStruct((_OUT_ROWS, _BLK), jnp.float32),
    scratch_types=[
        pltpu.VMEM((BATCH,), jnp.int32),
        pltpu.VMEM((_LISTCAP,), jnp.int32),
        pltpu.VMEM((_LISTCAP,), jnp.int32),
        pltpu.VMEM((_LISTCAP,), jnp.int32),
        pltpu.VMEM((_LISTCAP,), jnp.int32),
        pltpu.VMEM((_NBUF, PARAM_DIM, _BLK), jnp.float32),
        pltpu.VMEM((_NBUF, _LANE, _BLK), jnp.float32),
        pltpu.VMEM((_NBUF, _LANE), jnp.int32),
        pltpu.SemaphoreType.DMA,
        pltpu.SemaphoreType.DMA,
        pltpu.SemaphoreType.DMA,
        pltpu.SemaphoreType.DMA,
        pltpu.SemaphoreType.DMA,
        pltpu.SemaphoreType.DMA,
        pltpu.SemaphoreType.DMA,
        pltpu.SemaphoreType.DMA,
        pltpu.SemaphoreType.DMA,
    ],
    compiler_params=pltpu.CompilerParams(use_tc_tiling_on_sc=True, needs_layout_passes=False),
)
def _lookup_kernel(xs_hbm, tableT_hbm, out_hbm, x_v, eloc, ploc, me, mp,
                   slab, rows, posb, sem_x, ss0, ss1, ss2, ss3,
                   sw0, sw1, sw2, sw3):
    sem_s = [ss0, ss1, ss2, ss3]
    sem_w = [sw0, sw1, sw2, sw3]
    wid = lax.axis_index("s") * _NC + lax.axis_index("c")
    blk0 = wid * _BPW
    lo = blk0 * _BLK
    hi = lo + _BPW * _BLK
    iota = lax.iota(jnp.int32, _LANE)

    pltpu.async_copy(xs_hbm, x_v, sem_x).wait()

    def p1_body(j, n):
        evec = x_v[pl.ds(j * _LANE, _LANE)]
        m = (evec >= lo) & (evec < hi)
        pvec = iota + j * _LANE
        rank = plsc.cumsum(jnp.where(m, 1, 0))
        idxv = n + rank - 1
        plsc.store_scatter(eloc, [idxv], evec, mask=m)
        plsc.store_scatter(ploc, [idxv], pvec, mask=m)
        return n + jnp.sum(jnp.where(m, 1, 0))

    n_loc = lax.fori_loop(0, BATCH // _LANE, p1_body, jnp.int32(0))
    eloc[pl.ds(n_loc, _LANE)] = jnp.full((_LANE,), _SENTINEL, jnp.int32)
    nv = (n_loc + _LANE - 1) // _LANE

    def slab_start(b, r):
        blk = blk0 + b

        @pl.when((b < _BPW) & (blk < _NBLOCKS_FULL))
        def _():
            pltpu.async_copy(
                tableT_hbm.at[:, pl.ds(blk * _BLK, _BLK)], slab.at[r],
                sem_s[r],
            )

    for r in range(_NBUF):
        slab_start(jnp.int32(r), r)

    def process(b, r):
        blk = blk0 + b
        valid = (b < _BPW) & (blk < _NBLOCKS_FULL)
        dump = BATCH + (blk & (_NDUMP - 1))

        @pl.when(valid)
        def _():
            pltpu.make_async_copy(
                tableT_hbm.at[:, pl.ds(blk * _BLK, _BLK)], slab.at[r],
                sem_s[r],
            ).wait()

        def scan_body(j, k):
            evec = eloc[pl.ds(j * _LANE, _LANE)]
            pvec = ploc[pl.ds(j * _LANE, _LANE)]
            m = lax.shift_right_logical(evec, 7) == blk
            rank = plsc.cumsum(jnp.where(m, 1, 0))
            idxv = k + rank - 1
            plsc.store_scatter(me, [idxv], evec, mask=m)
            plsc.store_scatter(mp, [idxv], pvec, mask=m)
            return k + jnp.sum(jnp.where(m, 1, 0))

        nv_eff = jnp.where(valid, nv, 0)
        k = lax.fori_loop(0, nv_eff, scan_body, jnp.int32(0))

        def round_body(h, rcarry):
            @pl.when(h > 0)
            def _wait_prev():
                pltpu.make_async_copy(rows.at[r], out_hbm.at[posb.at[r]],
                                      sem_w[r]).wait()

            def ext_body(tt, pvec):
                t = h * _LANE + tt
                jv = (t // _LANE) * _LANE
                sel = iota == (t % _LANE)
                lvec = me[pl.ds(jv, _LANE)]
                pv = mp[pl.ds(jv, _LANE)]
                lane = jnp.sum(jnp.where(sel, lvec & (_BLK - 1), 0))
                p_t = jnp.sum(jnp.where(sel, pv, 0))
                for q in range(PARAM_DIM // _LANE):
                    didx = iota + q * _LANE
                    v = plsc.load_gather(slab.at[r],
                                         [didx, jnp.full((_LANE,), lane,
                                                         jnp.int32)])
                    plsc.store_scatter(rows.at[r],
                                       [jnp.full((_LANE,), tt, jnp.int32),
                                        didx], v)
                return jnp.where(iota == tt, p_t, pvec)

            kk = jnp.maximum(jnp.minimum(k - h * _LANE, _LANE), 0)
            pvec = lax.fori_loop(0, kk, ext_body,
                                 jnp.full((_LANE,), dump, jnp.int32))
            plsc.store_scatter(posb.at[r], [iota], pvec)
            pltpu.async_copy(rows.at[r], out_hbm.at[posb.at[r]], sem_w[r])
            return rcarry

        rounds = jnp.maximum((k + _LANE - 1) // _LANE, 1)
        lax.fori_loop(0, rounds, round_body, jnp.int32(0))
        slab_start(b + _NBUF, r)

    def step_body(step, carry):
        for r in range(_NBUF):
            @pl.when(step > 0)
            def _():
                pltpu.make_async_copy(rows.at[r], out_hbm.at[posb.at[r]],
                                      sem_w[r]).wait()

            process(step * _NBUF + r, r)
        return carry

    lax.fori_loop(0, _STEPS, step_body, jnp.int32(0))

    for r in range(_NBUF):
        pltpu.make_async_copy(rows.at[r], out_hbm.at[posb.at[r]],
                              sem_w[r]).wait()


def kernel(x, table):
    mask = x < _N_MAIN
    xs = jnp.where(mask, x, 0)
    out128 = _lookup_kernel(xs, table.T)
    main = out128[:BATCH, :PARAM_DIM]
    tail_ids = jnp.clip(x - _N_MAIN, 0, N_ROWS - _N_MAIN - 1)
    onehot = (tail_ids[:, None]
              == lax.broadcasted_iota(jnp.int32, (1, N_ROWS - _N_MAIN), 1)
              ).astype(jnp.float32)
    tail_rows = onehot @ table[_N_MAIN:]
    return jnp.where(mask[:, None], main, tail_rows)


# speedup vs baseline: 2.9734x; 1.0437x over previous
import functools

import jax
import jax.numpy as jnp
from jax import lax
from jax.experimental import pallas as pl
from jax.experimental.pallas import tpu as pltpu
from jax.experimental.pallas import tpu_sc as plsc

N_ROWS = 1000000
PARAM_DIM = 64
BATCH = 16384

_NC = 2
_NS = 16
_NW = _NC * _NS

_LANE = 16
_BLK = 128
_NBLOCKS_FULL = 7812
_N_MAIN = _NBLOCKS_FULL * _BLK
_BPW = 245
_NBUF = 8
_STEPS = (_BPW + _NBUF - 1) // _NBUF
_LISTCAP = 2048
_NSB = 8
_SBCAP = 512
_NDUMP = 512
_OUT_ROWS = BATCH + _NDUMP
_SENTINEL = 0x7FFFFF00

_mesh = plsc.VectorSubcoreMesh(core_axis_name="c", subcore_axis_name="s")


@functools.partial(
    pl.kernel,
    mesh=_mesh,
    out_type=jax.ShapeDtypeStruct((_OUT_ROWS, _BLK), jnp.float32),
    scratch_types=[
        pltpu.VMEM((BATCH,), jnp.int32),
        pltpu.VMEM((_LISTCAP,), jnp.int32),
        pltpu.VMEM((_LISTCAP,), jnp.int32),
        pltpu.VMEM((_NSB * _SBCAP,), jnp.int32),
        pltpu.VMEM((_NSB * _SBCAP,), jnp.int32),
        pltpu.VMEM((_LISTCAP,), jnp.int32),
        pltpu.VMEM((_LISTCAP,), jnp.int32),
        pltpu.VMEM((_NBUF, PARAM_DIM, _BLK), jnp.float32),
        pltpu.VMEM((_NBUF, _LANE, _BLK), jnp.float32),
        pltpu.VMEM((_NBUF, _LANE), jnp.int32),
        pltpu.SemaphoreType.DMA,
        pltpu.SemaphoreType.DMA,
        pltpu.SemaphoreType.DMA,
        pltpu.SemaphoreType.DMA,
        pltpu.SemaphoreType.DMA,
        pltpu.SemaphoreType.DMA,
        pltpu.SemaphoreType.DMA,
        pltpu.SemaphoreType.DMA,
        pltpu.SemaphoreType.DMA,
        pltpu.SemaphoreType.DMA,
        pltpu.SemaphoreType.DMA,
        pltpu.SemaphoreType.DMA,
        pltpu.SemaphoreType.DMA,
        pltpu.SemaphoreType.DMA,
        pltpu.SemaphoreType.DMA,
        pltpu.SemaphoreType.DMA,
        pltpu.SemaphoreType.DMA,
    ],
    compiler_params=pltpu.CompilerParams(use_tc_tiling_on_sc=True,
                                         needs_layout_passes=False),
)
def _lookup_kernel(xs_hbm, tableT_hbm, out_hbm, x_v, eloc, ploc, sbe, sbp,
                   me, mp, slab, rows, posb, sem_x,
                   ss0, ss1, ss2, ss3, ss4, ss5, ss6, ss7,
                   sw0, sw1, sw2, sw3, sw4, sw5, sw6, sw7):
    sem_s = [ss0, ss1, ss2, ss3, ss4, ss5, ss6, ss7]
    sem_w = [sw0, sw1, sw2, sw3, sw4, sw5, sw6, sw7]
    wid = lax.axis_index("s") * _NC + lax.axis_index("c")
    blk0 = wid * _BPW
    lo = blk0 * _BLK
    hi = lo + _BPW * _BLK
    iota = lax.iota(jnp.int32, _LANE)

    pltpu.async_copy(xs_hbm, x_v, sem_x).wait()

    def p1_body(j, n):
        evec = x_v[pl.ds(j * _LANE, _LANE)]
        m = (evec >= lo) & (evec < hi)
        pvec = iota + j * _LANE
        rank = plsc.cumsum(jnp.where(m, 1, 0))
        idxv = n + rank - 1
        plsc.store_scatter(eloc, [idxv], evec, mask=m)
        plsc.store_scatter(ploc, [idxv], pvec, mask=m)
        return n + jnp.sum(jnp.where(m, 1, 0))

    n_loc = lax.fori_loop(0, BATCH // _LANE, p1_body, jnp.int32(0))
    eloc[pl.ds(n_loc, _LANE)] = jnp.full((_LANE,), _SENTINEL, jnp.int32)
    nv = (n_loc + _LANE - 1) // _LANE

    def bin_body(j, cnts):
        evec = eloc[pl.ds(j * _LANE, _LANE)]
        pvec = ploc[pl.ds(j * _LANE, _LANE)]
        mv = evec != _SENTINEL
        sv = lax.shift_right_logical(evec - lo, 12)
        new = []
        for s in range(_NSB):
            m = mv & (sv == s)
            rank = plsc.cumsum(jnp.where(m, 1, 0))
            idxv = s * _SBCAP + cnts[s] + rank - 1
            plsc.store_scatter(sbe, [idxv], evec, mask=m)
            plsc.store_scatter(sbp, [idxv], pvec, mask=m)
            new.append(cnts[s] + jnp.sum(jnp.where(m, 1, 0)))
        return tuple(new)

    cnts = lax.fori_loop(0, nv, bin_body,
                         tuple(jnp.int32(0) for _ in range(_NSB)))
    sent = jnp.full((_LANE,), _SENTINEL, jnp.int32)
    nvs = []
    for s in range(_NSB):
        sbe[pl.ds(s * _SBCAP + cnts[s], _LANE)] = sent
        nvs.append((cnts[s] + _LANE - 1) // _LANE)

    def slab_start(b, r):
        blk = blk0 + b

        @pl.when((b < _BPW) & (blk < _NBLOCKS_FULL))
        def _():
            pltpu.async_copy(
                tableT_hbm.at[:, pl.ds(blk * _BLK, _BLK)], slab.at[r],
                sem_s[r],
            )

    for r in range(_NBUF):
        slab_start(jnp.int32(r), r)

    def process(b, r):
        blk = blk0 + b
        valid = (b < _BPW) & (blk < _NBLOCKS_FULL)
        dump = BATCH + (blk & (_NDUMP - 1))
        sb = lax.shift_right_logical(b, 5)
        nv_s = nvs[_NSB - 1]
        for s in range(_NSB - 1):
            nv_s = jnp.where(sb == s, nvs[s], nv_s)
        sb_base = sb * _SBCAP

        @pl.when(valid)
        def _():
            pltpu.make_async_copy(
                tableT_hbm.at[:, pl.ds(blk * _BLK, _BLK)], slab.at[r],
                sem_s[r],
            ).wait()

        def scan_body(j, k):
            evec = sbe[pl.ds(sb_base + j * _LANE, _LANE)]
            pvec = sbp[pl.ds(sb_base + j * _LANE, _LANE)]
            m = lax.shift_right_logical(evec, 7) == blk
            rank = plsc.cumsum(jnp.where(m, 1, 0))
            idxv = k + rank - 1
            plsc.store_scatter(me, [idxv], evec, mask=m)
            plsc.store_scatter(mp, [idxv], pvec, mask=m)
            return k + jnp.sum(jnp.where(m, 1, 0))

        nv_eff = jnp.where(valid, nv_s, 0)
        k = lax.fori_loop(0, nv_eff, scan_body, jnp.int32(0))

        def round_body(h, rcarry):
            @pl.when(h > 0)
            def _wait_prev():
                pltpu.make_async_copy(rows.at[r], out_hbm.at[posb.at[r]],
                                      sem_w[r]).wait()

            def ext_body(tt, pvec):
                t = h * _LANE + tt
                jv = (t // _LANE) * _LANE
                sel = iota == (t % _LANE)
                lvec = me[pl.ds(jv, _LANE)]
                pv = mp[pl.ds(jv, _LANE)]
                lane = jnp.sum(jnp.where(sel, lvec & (_BLK - 1), 0))
                p_t = jnp.sum(jnp.where(sel, pv, 0))
                for q in range(PARAM_DIM // _LANE):
                    didx = iota + q * _LANE
                    v = plsc.load_gather(slab.at[r],
                                         [didx, jnp.full((_LANE,), lane,
                                                         jnp.int32)])
                    plsc.store_scatter(rows.at[r],
                                       [jnp.full((_LANE,), tt, jnp.int32),
                                        didx], v)
                return jnp.where(iota == tt, p_t, pvec)

            kk = jnp.maximum(jnp.minimum(k - h * _LANE, _LANE), 0)
            pvec = lax.fori_loop(0, kk, ext_body,
                                 jnp.full((_LANE,), dump, jnp.int32))
            plsc.store_scatter(posb.at[r], [iota], pvec)
            pltpu.async_copy(rows.at[r], out_hbm.at[posb.at[r]], sem_w[r])
            return rcarry

        rounds = jnp.maximum((k + _LANE - 1) // _LANE, 1)
        lax.fori_loop(0, rounds, round_body, jnp.int32(0))
        slab_start(b + _NBUF, r)

    def step_body(step, carry):
        for r in range(_NBUF):
            @pl.when(step > 0)
            def _():
                pltpu.make_async_copy(rows.at[r], out_hbm.at[posb.at[r]],
                                      sem_w[r]).wait()

            process(step * _NBUF + r, r)
        return carry

    lax.fori_loop(0, _STEPS, step_body, jnp.int32(0))

    for r in range(_NBUF):
        pltpu.make_async_copy(rows.at[r], out_hbm.at[posb.at[r]],
                              sem_w[r]).wait()


def kernel(x, table):
    mask = x < _N_MAIN
    xs = jnp.where(mask, x, 0)
    out128 = _lookup_kernel(xs, table.T)
    main = out128[:BATCH, :PARAM_DIM]
    tail_ids = jnp.clip(x - _N_MAIN, 0, N_ROWS - _N_MAIN - 1)
    onehot = (tail_ids[:, None]
              == lax.broadcasted_iota(jnp.int32, (1, N_ROWS - _N_MAIN), 1)
              ).astype(jnp.float32)
    tail_rows = onehot @ table[_N_MAIN:]
    return jnp.where(mask[:, None], main, tail_rows)


# speedup vs baseline: 3.3664x; 1.1322x over previous
import functools

import jax
import jax.numpy as jnp
from jax import lax
from jax.experimental import pallas as pl
from jax.experimental.pallas import tpu as pltpu
from jax.experimental.pallas import tpu_sc as plsc

N_ROWS = 1000000
PARAM_DIM = 64
BATCH = 16384

_NC = 2
_NS = 16
_NW = _NC * _NS

_LANE = 16
_BLK = 256
_SHIFT = 8
_NBLOCKS_FULL = 3906
_N_MAIN = _NBLOCKS_FULL * _BLK
_BPW = 123
_NSBUF = 4
_NBUF = 8
_STEPS = (_BPW + _NBUF - 1) // _NBUF
_LISTCAP = 2048
_NSB = 8
_SBCAP = 512
_NDUMP = 512
_OUT_ROWS = BATCH + _NDUMP
_SENTINEL = 0x7FFFFF00

_mesh = plsc.VectorSubcoreMesh(core_axis_name="c", subcore_axis_name="s")


@functools.partial(
    pl.kernel,
    mesh=_mesh,
    out_type=jax.ShapeDtypeStruct((_OUT_ROWS, _BLK), jnp.float32),
    scratch_types=[
        pltpu.VMEM((BATCH,), jnp.int32),
        pltpu.VMEM((_LISTCAP,), jnp.int32),
        pltpu.VMEM((_LISTCAP,), jnp.int32),
        pltpu.VMEM((_NSB * _SBCAP,), jnp.int32),
        pltpu.VMEM((_NSB * _SBCAP,), jnp.int32),
        pltpu.VMEM((_SBCAP,), jnp.int32),
        pltpu.VMEM((_SBCAP,), jnp.int32),
        pltpu.VMEM((_NSBUF, PARAM_DIM, _BLK), jnp.float32),
        pltpu.VMEM((_NBUF, _LANE, _BLK), jnp.float32),
        pltpu.VMEM((_NBUF, _LANE), jnp.int32),
        pltpu.SemaphoreType.DMA,
        pltpu.SemaphoreType.DMA,
        pltpu.SemaphoreType.DMA,
        pltpu.SemaphoreType.DMA,
        pltpu.SemaphoreType.DMA,
        pltpu.SemaphoreType.DMA,
        pltpu.SemaphoreType.DMA,
        pltpu.SemaphoreType.DMA,
        pltpu.SemaphoreType.DMA,
        pltpu.SemaphoreType.DMA,
        pltpu.SemaphoreType.DMA,
        pltpu.SemaphoreType.DMA,
        pltpu.SemaphoreType.DMA,
    ],
    compiler_params=pltpu.CompilerParams(use_tc_tiling_on_sc=True,
                                         needs_layout_passes=False),
)
def _lookup_kernel(xs_hbm, tableT_hbm, out_hbm, x_v, eloc, ploc, sbe, sbp,
                   me, mp, slab, rows, posb, sem_x,
                   ss0, ss1, ss2, ss3,
                   sw0, sw1, sw2, sw3, sw4, sw5, sw6, sw7):
    sem_s = [ss0, ss1, ss2, ss3]
    sem_w = [sw0, sw1, sw2, sw3, sw4, sw5, sw6, sw7]
    wid = lax.axis_index("s") * _NC + lax.axis_index("c")
    blk0 = wid * _BPW
    lo = blk0 * _BLK
    hi = lo + _BPW * _BLK
    iota = lax.iota(jnp.int32, _LANE)

    def slab_start(b, r):
        blk = blk0 + b

        @pl.when((b < _BPW) & (blk < _NBLOCKS_FULL))
        def _():
            pltpu.async_copy(
                tableT_hbm.at[:, pl.ds(blk * _BLK, _BLK)], slab.at[r],
                sem_s[r],
            )

    for r in range(_NSBUF):
        slab_start(jnp.int32(r), r)

    pltpu.async_copy(xs_hbm, x_v, sem_x).wait()

    def p1_body(j, n):
        evec = x_v[pl.ds(j * _LANE, _LANE)]
        m = (evec >= lo) & (evec < hi)
        pvec = iota + j * _LANE
        rank = plsc.cumsum(jnp.where(m, 1, 0))
        idxv = n + rank - 1
        plsc.store_scatter(eloc, [idxv], evec, mask=m)
        plsc.store_scatter(ploc, [idxv], pvec, mask=m)
        return n + jnp.sum(jnp.where(m, 1, 0))

    n_loc = lax.fori_loop(0, BATCH // _LANE, p1_body, jnp.int32(0))
    eloc[pl.ds(n_loc, _LANE)] = jnp.full((_LANE,), _SENTINEL, jnp.int32)
    nv = (n_loc + _LANE - 1) // _LANE

    def bin_body(j, cnts):
        evec = eloc[pl.ds(j * _LANE, _LANE)]
        pvec = ploc[pl.ds(j * _LANE, _LANE)]
        mv = evec != _SENTINEL
        sv = lax.shift_right_logical(evec - lo, 12)
        new = []
        for s in range(_NSB):
            m = mv & (sv == s)
            rank = plsc.cumsum(jnp.where(m, 1, 0))
            idxv = s * _SBCAP + cnts[s] + rank - 1
            plsc.store_scatter(sbe, [idxv], evec, mask=m)
            plsc.store_scatter(sbp, [idxv], pvec, mask=m)
            new.append(cnts[s] + jnp.sum(jnp.where(m, 1, 0)))
        return tuple(new)

    cnts = lax.fori_loop(0, nv, bin_body,
                         tuple(jnp.int32(0) for _ in range(_NSB)))
    sent = jnp.full((_LANE,), _SENTINEL, jnp.int32)
    nvs = []
    for s in range(_NSB):
        sbe[pl.ds(s * _SBCAP + cnts[s], _LANE)] = sent
        nvs.append((cnts[s] + _LANE - 1) // _LANE)

    def process(b, r, rs):
        blk = blk0 + b
        valid = (b < _BPW) & (blk < _NBLOCKS_FULL)
        dump = BATCH + (blk & (_NDUMP - 1))
        sb = lax.shift_right_logical(b, 4)
        nv_s = nvs[_NSB - 1]
        for s in range(_NSB - 1):
            nv_s = jnp.where(sb == s, nvs[s], nv_s)
        sb_base = sb * _SBCAP

        @pl.when(valid)
        def _():
            pltpu.make_async_copy(
                tableT_hbm.at[:, pl.ds(blk * _BLK, _BLK)], slab.at[rs],
                sem_s[rs],
            ).wait()

        def scan_body(j, k):
            evec = sbe[pl.ds(sb_base + j * _LANE, _LANE)]
            pvec = sbp[pl.ds(sb_base + j * _LANE, _LANE)]
            m = lax.shift_right_logical(evec, _SHIFT) == blk
            rank = plsc.cumsum(jnp.where(m, 1, 0))
            idxv = k + rank - 1
            plsc.store_scatter(me, [idxv], evec, mask=m)
            plsc.store_scatter(mp, [idxv], pvec, mask=m)
            return k + jnp.sum(jnp.where(m, 1, 0))

        nv_eff = jnp.where(valid, nv_s, 0)
        k = lax.fori_loop(0, nv_eff, scan_body, jnp.int32(0))

        def round_body(h, rcarry):
            @pl.when(h > 0)
            def _wait_prev():
                pltpu.make_async_copy(rows.at[r], out_hbm.at[posb.at[r]],
                                      sem_w[r]).wait()

            def ext_body(tt, pvec):
                t = h * _LANE + tt
                jv = (t // _LANE) * _LANE
                sel = iota == (t % _LANE)
                lvec = me[pl.ds(jv, _LANE)]
                pv = mp[pl.ds(jv, _LANE)]
                lane = jnp.sum(jnp.where(sel, lvec & (_BLK - 1), 0))
                p_t = jnp.sum(jnp.where(sel, pv, 0))
                for q in range(PARAM_DIM // _LANE):
                    didx = iota + q * _LANE
                    v = plsc.load_gather(slab.at[rs],
                                         [didx, jnp.full((_LANE,), lane,
                                                         jnp.int32)])
                    plsc.store_scatter(rows.at[r],
                                       [jnp.full((_LANE,), tt, jnp.int32),
                                        didx], v)
                return jnp.where(iota == tt, p_t, pvec)

            kk = jnp.maximum(jnp.minimum(k - h * _LANE, _LANE), 0)
            pvec = lax.fori_loop(0, kk, ext_body,
                                 jnp.full((_LANE,), dump, jnp.int32))
            plsc.store_scatter(posb.at[r], [iota], pvec)
            pltpu.async_copy(rows.at[r], out_hbm.at[posb.at[r]], sem_w[r])
            return rcarry

        rounds = jnp.maximum((k + _LANE - 1) // _LANE, 1)
        lax.fori_loop(0, rounds, round_body, jnp.int32(0))
        slab_start(b + _NSBUF, rs)

    def step_body(step, carry):
        for r in range(_NBUF):
            @pl.when(step > 0)
            def _():
                pltpu.make_async_copy(rows.at[r], out_hbm.at[posb.at[r]],
                                      sem_w[r]).wait()

            process(step * _NBUF + r, r, r % _NSBUF)
        return carry

    lax.fori_loop(0, _STEPS, step_body, jnp.int32(0))

    for r in range(_NBUF):
        pltpu.make_async_copy(rows.at[r], out_hbm.at[posb.at[r]],
                              sem_w[r]).wait()


def kernel(x, table):
    mask = x < _N_MAIN
    xs = jnp.where(mask, x, 0)
    out128 = _lookup_kernel(xs, table.T)
    main = out128[:BATCH, :PARAM_DIM]
    tail_ids = jnp.clip(x - _N_MAIN, 0, N_ROWS - _N_MAIN - 1)
    onehot = (tail_ids[:, None]
              == lax.broadcasted_iota(jnp.int32, (1, N_ROWS - _N_MAIN), 1)
              ).astype(jnp.float32)
    tail_rows = onehot @ table[_N_MAIN:]
    return jnp.where(mask[:, None], main, tail_rows)


# speedup vs baseline: 3.6953x; 1.0977x over previous
import functools

import jax
import jax.numpy as jnp
from jax import lax
from jax.experimental import pallas as pl
from jax.experimental.pallas import tpu as pltpu
from jax.experimental.pallas import tpu_sc as plsc

N_ROWS = 1000000
PARAM_DIM = 64
BATCH = 16384

_NC = 2
_NS = 16
_NW = _NC * _NS

_LANE = 16
_BLK = 512
_SHIFT = 9
_NBLOCKS_FULL = 1953
_N_MAIN = _NBLOCKS_FULL * _BLK
_BPW = 62
_NSBUF = 2
_NBUF = 8
_STEPS = (_BPW + _NBUF - 1) // _NBUF
_LISTCAP = 2048
_NSB = 8
_SBCAP = 512
_NDUMP = 512
_OUT_ROWS = BATCH + _NDUMP
_OUTW = 128
_SENTINEL = 0x7FFFFF00

_mesh = plsc.VectorSubcoreMesh(core_axis_name="c", subcore_axis_name="s")


@functools.partial(
    pl.kernel,
    mesh=_mesh,
    out_type=jax.ShapeDtypeStruct((_OUT_ROWS, _OUTW), jnp.float32),
    scratch_types=[
        pltpu.VMEM((BATCH,), jnp.int32),
        pltpu.VMEM((_LISTCAP,), jnp.int32),
        pltpu.VMEM((_LISTCAP,), jnp.int32),
        pltpu.VMEM((_NSB * _SBCAP,), jnp.int32),
        pltpu.VMEM((_NSB * _SBCAP,), jnp.int32),
        pltpu.VMEM((_SBCAP,), jnp.int32),
        pltpu.VMEM((_SBCAP,), jnp.int32),
        pltpu.VMEM((_NSBUF, PARAM_DIM, _BLK), jnp.float32),
        pltpu.VMEM((_NBUF, _LANE, _OUTW), jnp.float32),
        pltpu.VMEM((_NBUF, _LANE), jnp.int32),
        pltpu.SemaphoreType.DMA,
        pltpu.SemaphoreType.DMA,
        pltpu.SemaphoreType.DMA,
        pltpu.SemaphoreType.DMA,
        pltpu.SemaphoreType.DMA,
        pltpu.SemaphoreType.DMA,
        pltpu.SemaphoreType.DMA,
        pltpu.SemaphoreType.DMA,
        pltpu.SemaphoreType.DMA,
        pltpu.SemaphoreType.DMA,
        pltpu.SemaphoreType.DMA,
    ],
    compiler_params=pltpu.CompilerParams(use_tc_tiling_on_sc=True,
                                         needs_layout_passes=False),
)
def _lookup_kernel(xs_hbm, tableT_hbm, out_hbm, x_v, eloc, ploc, sbe, sbp,
                   me, mp, slab, rows, posb, sem_x,
                   ss0, ss1,
                   sw0, sw1, sw2, sw3, sw4, sw5, sw6, sw7):
    sem_s = [ss0, ss1]
    sem_w = [sw0, sw1, sw2, sw3, sw4, sw5, sw6, sw7]
    wid = lax.axis_index("s") * _NC + lax.axis_index("c")
    blk0 = wid * _BPW
    lo = blk0 * _BLK
    hi = lo + _BPW * _BLK
    iota = lax.iota(jnp.int32, _LANE)

    def slab_start(b, r):
        blk = blk0 + b

        @pl.when((b < _BPW) & (blk < _NBLOCKS_FULL))
        def _():
            pltpu.async_copy(
                tableT_hbm.at[:, pl.ds(blk * _BLK, _BLK)], slab.at[r],
                sem_s[r],
            )

    for r in range(_NSBUF):
        slab_start(jnp.int32(r), r)

    pltpu.async_copy(xs_hbm, x_v, sem_x).wait()

    def p1_body(j, n):
        evec = x_v[pl.ds(j * _LANE, _LANE)]
        m = (evec >= lo) & (evec < hi)
        pvec = iota + j * _LANE
        rank = plsc.cumsum(jnp.where(m, 1, 0))
        idxv = n + rank - 1
        plsc.store_scatter(eloc, [idxv], evec, mask=m)
        plsc.store_scatter(ploc, [idxv], pvec, mask=m)
        return n + jnp.sum(jnp.where(m, 1, 0))

    n_loc = lax.fori_loop(0, BATCH // _LANE, p1_body, jnp.int32(0))
    eloc[pl.ds(n_loc, _LANE)] = jnp.full((_LANE,), _SENTINEL, jnp.int32)
    nv = (n_loc + _LANE - 1) // _LANE

    def bin_body(j, cnts):
        evec = eloc[pl.ds(j * _LANE, _LANE)]
        pvec = ploc[pl.ds(j * _LANE, _LANE)]
        mv = evec != _SENTINEL
        sv = lax.shift_right_logical(evec - lo, 12)
        new = []
        for s in range(_NSB):
            m = mv & (sv == s)
            rank = plsc.cumsum(jnp.where(m, 1, 0))
            idxv = s * _SBCAP + cnts[s] + rank - 1
            plsc.store_scatter(sbe, [idxv], evec, mask=m)
            plsc.store_scatter(sbp, [idxv], pvec, mask=m)
            new.append(cnts[s] + jnp.sum(jnp.where(m, 1, 0)))
        return tuple(new)

    cnts = lax.fori_loop(0, nv, bin_body,
                         tuple(jnp.int32(0) for _ in range(_NSB)))
    sent = jnp.full((_LANE,), _SENTINEL, jnp.int32)
    nvs = []
    for s in range(_NSB):
        sbe[pl.ds(s * _SBCAP + cnts[s], _LANE)] = sent
        nvs.append((cnts[s] + _LANE - 1) // _LANE)

    def process(b, r, rs):
        blk = blk0 + b
        valid = (b < _BPW) & (blk < _NBLOCKS_FULL)
        dump = BATCH + (blk & (_NDUMP - 1))
        sb = lax.shift_right_logical(b, 3)
        nv_s = nvs[_NSB - 1]
        for s in range(_NSB - 1):
            nv_s = jnp.where(sb == s, nvs[s], nv_s)
        sb_base = sb * _SBCAP

        @pl.when(valid)
        def _():
            pltpu.make_async_copy(
                tableT_hbm.at[:, pl.ds(blk * _BLK, _BLK)], slab.at[rs],
                sem_s[rs],
            ).wait()

        def scan_body(j, k):
            evec = sbe[pl.ds(sb_base + j * _LANE, _LANE)]
            pvec = sbp[pl.ds(sb_base + j * _LANE, _LANE)]
            m = lax.shift_right_logical(evec, _SHIFT) == blk
            rank = plsc.cumsum(jnp.where(m, 1, 0))
            idxv = k + rank - 1
            plsc.store_scatter(me, [idxv], evec, mask=m)
            plsc.store_scatter(mp, [idxv], pvec, mask=m)
            return k + jnp.sum(jnp.where(m, 1, 0))

        nv_eff = jnp.where(valid, nv_s, 0)
        k = lax.fori_loop(0, nv_eff, scan_body, jnp.int32(0))

        def round_body(h, rcarry):
            @pl.when(h > 0)
            def _wait_prev():
                pltpu.make_async_copy(rows.at[r], out_hbm.at[posb.at[r]],
                                      sem_w[r]).wait()

            def ext_body(tt, pvec):
                t = h * _LANE + tt
                jv = (t // _LANE) * _LANE
                sel = iota == (t % _LANE)
                lvec = me[pl.ds(jv, _LANE)]
                pv = mp[pl.ds(jv, _LANE)]
                lane = jnp.sum(jnp.where(sel, lvec & (_BLK - 1), 0))
                p_t = jnp.sum(jnp.where(sel, pv, 0))
                for q in range(PARAM_DIM // _LANE):
                    didx = iota + q * _LANE
                    v = plsc.load_gather(slab.at[rs],
                                         [didx, jnp.full((_LANE,), lane,
                                                         jnp.int32)])
                    plsc.store_scatter(rows.at[r],
                                       [jnp.full((_LANE,), tt, jnp.int32),
                                        didx], v)
                return jnp.where(iota == tt, p_t, pvec)

            kk = jnp.maximum(jnp.minimum(k - h * _LANE, _LANE), 0)
            pvec = lax.fori_loop(0, kk, ext_body,
                                 jnp.full((_LANE,), dump, jnp.int32))
            plsc.store_scatter(posb.at[r], [iota], pvec)
            pltpu.async_copy(rows.at[r], out_hbm.at[posb.at[r]], sem_w[r])
            return rcarry

        rounds = jnp.maximum((k + _LANE - 1) // _LANE, 1)
        lax.fori_loop(0, rounds, round_body, jnp.int32(0))
        slab_start(b + _NSBUF, rs)

    def step_body(step, carry):
        for r in range(_NBUF):
            @pl.when(step > 0)
            def _():
                pltpu.make_async_copy(rows.at[r], out_hbm.at[posb.at[r]],
                                      sem_w[r]).wait()

            process(step * _NBUF + r, r, r % _NSBUF)
        return carry

    lax.fori_loop(0, _STEPS, step_body, jnp.int32(0))

    for r in range(_NBUF):
        pltpu.make_async_copy(rows.at[r], out_hbm.at[posb.at[r]],
                              sem_w[r]).wait()


def kernel(x, table):
    mask = x < _N_MAIN
    xs = jnp.where(mask, x, 0)
    out128 = _lookup_kernel(xs, table.T)
    main = out128[:BATCH, :PARAM_DIM]
    tail_ids = jnp.clip(x - _N_MAIN, 0, N_ROWS - _N_MAIN - 1)
    onehot = (tail_ids[:, None]
              == lax.broadcasted_iota(jnp.int32, (1, N_ROWS - _N_MAIN), 1)
              ).astype(jnp.float32)
    tail_rows = onehot @ table[_N_MAIN:]
    return jnp.where(mask[:, None], main, tail_rows)
